# baseline (speedup 1.0000x reference)
import functools

import jax
import jax.numpy as jnp
from jax import lax
from jax.experimental import pallas as pl
from jax.experimental.pallas import tpu as pltpu
from jax.experimental.pallas import tpu_sc as plsc

N = 10000
E = 320000
D = 128
H = D // 2

NS = 16
CHUNK = 40
EPS = E // NS
NCHUNK = EPS // CHUNK
RPT = 624
RTAIL = N - NS * RPT

_f32 = jnp.float32


def _zero_vmem_rows(buf, nrows, width):
    z = jnp.zeros((16,), jnp.float32)

    def row(e, carry):
        for g in range(width // 16):
            buf[e, pl.ds(g * 16, 16)] = z
        return carry

    lax.fori_loop(0, nrows, row, 0)


def _acc_zero(s, zbuf, acc, nrows):
    rpt = (nrows // NS) // 8 * 8
    rtail = nrows - NS * rpt
    b = s * rpt
    for k in range(rpt // CHUNK):
        pltpu.sync_copy(
            zbuf.at[pl.ds(0, CHUNK)],
            acc.at[pl.ds(pl.multiple_of(b + k * CHUNK, 8), CHUNK)])
    rem = rpt - (rpt // CHUNK) * CHUNK
    if rem:
        pltpu.sync_copy(
            zbuf.at[pl.ds(0, rem)],
            acc.at[pl.ds(pl.multiple_of(b + rpt - rem, 8), rem)])

    if rtail:
        @pl.when(s == 0)
        def _tail():
            pltpu.sync_copy(zbuf.at[pl.ds(0, rtail)],
                            acc.at[pl.ds(pl.multiple_of(NS * rpt, 8), rtail)])


def _acc_copy(s, src, dst, nrows, dst_base=0):
    rpt = (nrows // NS) // 8 * 8
    rtail = nrows - NS * rpt
    b = s * rpt
    pltpu.sync_copy(src.at[pl.ds(pl.multiple_of(b, 8), rpt)],
                    dst.at[pl.ds(pl.multiple_of(dst_base + b, 8), rpt)])

    if rtail:
        @pl.when(s == 0)
        def _tail():
            t = NS * rpt
            pltpu.sync_copy(src.at[pl.ds(pl.multiple_of(t, 8), rtail)],
                            dst.at[pl.ds(pl.multiple_of(dst_base + t, 8), rtail)])


def _tc1_body(ft_ref, fs_ref, wnt_ref, w1_ref, w2_ref, bnt_ref, batt_ref,
              tab1_ref):
    ft = ft_ref[...]
    fs = fs_ref[...]
    t_tok = jnp.dot(ft, wnt_ref[...], preferred_element_type=_f32) + bnt_ref[...]
    t_srl = jnp.dot(fs, wnt_ref[...], preferred_element_type=_f32) + bnt_ref[...]
    a_src = jnp.dot(t_tok, w1_ref[...], preferred_element_type=_f32)
    a_dst = jnp.dot(t_srl, w2_ref[...], preferred_element_type=_f32) + batt_ref[...]
    zpad = jnp.zeros_like(a_dst[:, :H])
    for c in range(2):
        f0 = c * H
        tab1_ref[c, 0] = jnp.concatenate(
            [a_src[:, f0:f0 + H], t_tok[:, f0:f0 + H]], axis=1)
        tab1_ref[c, 1] = jnp.concatenate([a_dst[:, f0:f0 + H], zpad], axis=1)


def _tc1(ft, fs, wntT, w1T, w2T, bnt, batt):
    blk = 1000
    grid = (N // blk,)
    full = lambda shape: pl.BlockSpec(shape, lambda i: (0,) * len(shape))
    return pl.pallas_call(
        _tc1_body,
        grid=grid,
        in_specs=[
            pl.BlockSpec((blk, D), lambda i: (i, 0)),
            pl.BlockSpec((blk, D), lambda i: (i, 0)),
            full((D, D)), full((D, D)), full((D, D)),
            full((1, D)), full((1, D)),
        ],
        out_specs=pl.BlockSpec((2, 2, blk, D), lambda i: (0, 0, i, 0)),
        out_shape=jax.ShapeDtypeStruct((2, 2, N, D), _f32),
    )(ft, fs, wntT, w1T, w2T, bnt, batt)


def _sc_edge_pass(compute_edge, w_tab, acc_rows, parity):
    def body(*refs):
        if parity:
            (cidx, edges, pref, tab, out,
             bidx, didx, pbuf, grows, orows, acc, sem) = refs
        else:
            cidx, edges, tab, out, bidx, didx, grows, orows, acc, sem = refs
            pbuf = None
        c = lax.axis_index("c")
        s = lax.axis_index("s")
        _zero_vmem_rows(orows, CHUNK, D)
        _acc_zero(s, orows, acc, acc_rows)
        plsc.subcore_barrier()

        def chunk(j, carry):
            jg = s * NCHUNK + j
            pltpu.sync_copy(
                cidx.at[pl.ds(pl.multiple_of(jg * 2 * CHUNK, 8), 2 * CHUNK)],
                bidx)
            pltpu.sync_copy(
                edges.at[pl.ds(pl.multiple_of(E + jg * CHUNK, 8), CHUNK)],
                didx)
            if parity:
                pltpu.sync_copy(
                    pref.at[pl.ds(pl.multiple_of(jg * CHUNK, 8), CHUNK)],
                    pbuf)

            pltpu.async_copy(tab.at[c].at[bidx], grows, sem).wait()

            @plsc.parallel_loop(0, CHUNK, unroll=8)
            def edge(e):
                compute_edge(grows, orows, e, pbuf)
            pltpu.sync_copy(orows, acc.at[didx], add=True)
            return carry

        lax.fori_loop(0, NCHUNK, chunk, 0)
        plsc.subcore_barrier()
        _acc_copy(s, acc, out, acc_rows, dst_base=c * acc_rows)

    mesh = plsc.VectorSubcoreMesh(core_axis_name="c", subcore_axis_name="s")
    scratch = [
        pltpu.VMEM((2 * CHUNK,), jnp.int32),
        pltpu.VMEM((CHUNK,), jnp.int32),
    ]
    if parity:
        scratch.append(pltpu.VMEM((CHUNK, 16), _f32))
    scratch += [
        pltpu.VMEM((2 * CHUNK, w_tab), _f32),
        pltpu.VMEM((CHUNK, D), _f32),
        pltpu.VMEM_SHARED((acc_rows, D), _f32),
        pltpu.SemaphoreType.DMA,
    ]
    return functools.partial(
        pl.kernel,
        mesh=mesh,
        out_type=jax.ShapeDtypeStruct((2 * acc_rows, D), _f32),
        scratch_types=scratch,
    )(body)


def _sc1_edge(grows, orows, e, pbuf):
    for g in range(H // 16):
        sl = pl.ds(g * 16, 16)
        a = grows[e, sl] + grows[CHUNK + e, sl]
        a = jnp.maximum(a, a * 0.01)
        ex = jnp.exp(a)
        orows[e, sl] = ex * grows[e, pl.ds(H + g * 16, 16)]
        orows[e, pl.ds(H + g * 16, 16)] = ex


def _sc2_edge(grows, orows, e, pbuf):
    p = pbuf[e]
    q = 1.0 - p
    for g in range(H // 16):
        sl = pl.ds(g * 16, 16)
        ir = grows[e, sl]
        hr = grows[CHUNK + e, sl]
        iz = grows[e, pl.ds(H + g * 16, 16)]
        hz = grows[CHUNK + e, pl.ds(H + g * 16, 16)]
        inn = grows[e, pl.ds(2 * H + g * 16, 16)]
        hn = grows[CHUNK + e, pl.ds(2 * H + g * 16, 16)]
        hh = grows[CHUNK + e, pl.ds(3 * H + g * 16, 16)]
        r = 1.0 / (1.0 + jnp.exp(-(ir + hr)))
        z = 1.0 / (1.0 + jnp.exp(-(iz + hz)))
        x = inn + r * hn
        n = 1.0 - 2.0 / (jnp.exp(2.0 * x) + 1.0)
        m = n + z * (hh - n)
        orows[e, sl] = m * q
        orows[e, pl.ds(H + g * 16, 16)] = m * p


def _tc2_body(acc_ref, fs_ref, ft_ref, wih_ref, whh_ref, bih_ref, bhh_ref,
              hsrl_ref, tab2_ref):
    num = jnp.concatenate([acc_ref[0][:, :H], acc_ref[1][:, :H]], axis=1)
    den = jnp.concatenate([acc_ref[0][:, H:], acc_ref[1][:, H:]], axis=1)
    h = jnp.where(den > 0.0, num / den, fs_ref[...])
    hsrl_ref[...] = h
    ft = ft_ref[...]
    gi = jnp.dot(h, wih_ref[...], preferred_element_type=_f32) + bih_ref[...]
    gh = jnp.dot(ft, whh_ref[...], preferred_element_type=_f32) + bhh_ref[...]
    zpad = jnp.zeros_like(h[:, :H])
    for c in range(2):
        f0 = c * H
        tab2_ref[c, 0] = jnp.concatenate(
            [gi[:, f0:f0 + H], gi[:, D + f0:D + f0 + H],
             gi[:, 2 * D + f0:2 * D + f0 + H], zpad], axis=1)
        tab2_ref[c, 1] = jnp.concatenate(
            [gh[:, f0:f0 + H], gh[:, D + f0:D + f0 + H],
             gh[:, 2 * D + f0:2 * D + f0 + H], ft[:, f0:f0 + H]], axis=1)


def _tc2(acc1, fs, ft, wihT, whhT, bih, bhh):
    blk = 1000
    grid = (N // blk,)
    full = lambda shape: pl.BlockSpec(shape, lambda i: (0,) * len(shape))
    return pl.pallas_call(
        _tc2_body,
        grid=grid,
        in_specs=[
            pl.BlockSpec((2, blk, D), lambda i: (0, i, 0)),
            pl.BlockSpec((blk, D), lambda i: (i, 0)),
            pl.BlockSpec((blk, D), lambda i: (i, 0)),
            full((D, 3 * D)), full((D, 3 * D)),
            full((1, 3 * D)), full((1, 3 * D)),
        ],
        out_specs=[
            pl.BlockSpec((blk, D), lambda i: (i, 0)),
            pl.BlockSpec((2, 2, blk, 4 * H), lambda i: (0, 0, i, 0)),
        ],
        out_shape=[
            jax.ShapeDtypeStruct((N, D), _f32),
            jax.ShapeDtypeStruct((2, 2, N, 4 * H), _f32),
        ],
    )(acc1, fs, ft, wihT, whhT, bih, bhh)


def _tc3_body(acc_ref, wih_ref, whh_ref, bih_ref, bhh_ref, out_ref):
    h = jnp.concatenate([acc_ref[0], acc_ref[1]], axis=1)
    gi = jnp.dot(h, wih_ref[...], preferred_element_type=_f32) + bih_ref[...]
    gh = jnp.dot(h, whh_ref[...], preferred_element_type=_f32) + bhh_ref[...]
    r = jax.nn.sigmoid(gi[:, :D] + gh[:, :D])
    z = jax.nn.sigmoid(gi[:, D:2 * D] + gh[:, D:2 * D])
    n = jnp.tanh(gi[:, 2 * D:] + r * gh[:, 2 * D:])
    out_ref[...] = (1.0 - z) * n + z * h


def _tc3(acc2, wihT, whhT, bih, bhh):
    blk = 1000
    grid = (N // blk,)
    full = lambda shape: pl.BlockSpec(shape, lambda i: (0,) * len(shape))
    return pl.pallas_call(
        _tc3_body,
        grid=grid,
        in_specs=[
            pl.BlockSpec((2, blk, H), lambda i: (0, i, 0)),
            full((D, 3 * D)), full((D, 3 * D)),
            full((1, 3 * D)), full((1, 3 * D)),
        ],
        out_specs=pl.BlockSpec((blk, D), lambda i: (i, 0)),
        out_shape=jax.ShapeDtypeStruct((N, D), _f32),
    )(acc2, wihT, whhT, bih, bhh)


def _combined_idx(edges_i32):
    src = edges_i32[0].reshape(-1, CHUNK)
    dst = edges_i32[1].reshape(-1, CHUNK) + N
    return jnp.concatenate([src, dst], axis=1).reshape(-1)


def kernel(feat_tok, feat_srl, W_node_trans, b_node_trans, W_node_att,
           b_node_att, W_ih, W_hh, b_ih, b_hh, edge_tok2srl, edge_srl2tok):
    ft = feat_tok.astype(_f32)
    fs = feat_srl.astype(_f32)
    wntT = W_node_trans.astype(_f32).T
    w1T = W_node_att.astype(_f32)[:, :D].T
    w2T = W_node_att.astype(_f32)[:, D:].T
    wihT = W_ih.astype(_f32).T
    whhT = W_hh.astype(_f32).T
    bnt = b_node_trans.astype(_f32).reshape(1, D)
    batt = b_node_att.astype(_f32).reshape(1, D)
    bih = b_ih.astype(_f32).reshape(1, 3 * D)
    bhh = b_hh.astype(_f32).reshape(1, 3 * D)
    e1 = edge_tok2srl.astype(jnp.int32)
    e2 = edge_srl2tok.astype(jnp.int32)

    tab1 = _tc1(ft, fs, wntT, w1T, w2T, bnt, batt)
    sc1 = _sc_edge_pass(_sc1_edge, D, N, parity=False)
    acc1 = sc1(_combined_idx(e1), e1.reshape(-1), tab1.reshape(2, 2 * N, D))

    h_srl, tab2 = _tc2(acc1.reshape(2, N, D), fs, ft, wihT, whhT, bih, bhh)
    sc2 = _sc_edge_pass(_sc2_edge, 4 * H, N // 2, parity=True)
    e2_flat = jnp.concatenate([e2[0], e2[1] >> 1])
    par = jnp.broadcast_to((e2[1] & 1).astype(_f32)[:, None], (E, 16))
    acc2 = sc2(_combined_idx(e2), e2_flat, par, tab2.reshape(2, 2 * N, 4 * H))

    h_tok = _tc3(acc2.reshape(2, N, H), wihT, whhT, bih, bhh)
    return (h_tok, h_srl)


# speedup vs baseline: 1.7309x; 1.7309x over previous
import functools

import jax
import jax.numpy as jnp
from jax import lax
from jax.experimental import pallas as pl
from jax.experimental.pallas import tpu as pltpu
from jax.experimental.pallas import tpu_sc as plsc

N = 10000
E = 320000
D = 128
H = D // 2

NS = 16
CHUNK = 40
EPS = E // NS
NCHUNK = EPS // CHUNK
RPT = 624
RTAIL = N - NS * RPT

_f32 = jnp.float32


def _zero_vmem_rows(buf, nrows, width):
    z = jnp.zeros((16,), jnp.float32)

    def row(e, carry):
        for g in range(width // 16):
            buf[e, pl.ds(g * 16, 16)] = z
        return carry

    lax.fori_loop(0, nrows, row, 0)


def _acc_zero(s, zbuf, acc, nrows):
    rpt = (nrows // NS) // 8 * 8
    rtail = nrows - NS * rpt
    b = s * rpt
    for k in range(rpt // CHUNK):
        pltpu.sync_copy(
            zbuf.at[pl.ds(0, CHUNK)],
            acc.at[pl.ds(pl.multiple_of(b + k * CHUNK, 8), CHUNK)])
    rem = rpt - (rpt // CHUNK) * CHUNK
    if rem:
        pltpu.sync_copy(
            zbuf.at[pl.ds(0, rem)],
            acc.at[pl.ds(pl.multiple_of(b + rpt - rem, 8), rem)])

    if rtail:
        @pl.when(s == 0)
        def _tail():
            pltpu.sync_copy(zbuf.at[pl.ds(0, rtail)],
                            acc.at[pl.ds(pl.multiple_of(NS * rpt, 8), rtail)])


def _acc_copy(s, src, dst, nrows, dst_base=0):
    rpt = (nrows // NS) // 8 * 8
    rtail = nrows - NS * rpt
    b = s * rpt
    pltpu.sync_copy(src.at[pl.ds(pl.multiple_of(b, 8), rpt)],
                    dst.at[pl.ds(pl.multiple_of(dst_base + b, 8), rpt)])

    if rtail:
        @pl.when(s == 0)
        def _tail():
            t = NS * rpt
            pltpu.sync_copy(src.at[pl.ds(pl.multiple_of(t, 8), rtail)],
                            dst.at[pl.ds(pl.multiple_of(dst_base + t, 8), rtail)])


def _tc1_body(ft_ref, fs_ref, wnt_ref, w1_ref, w2_ref, bnt_ref, batt_ref,
              tab1_ref):
    ft = ft_ref[...]
    fs = fs_ref[...]
    t_tok = jnp.dot(ft, wnt_ref[...], preferred_element_type=_f32) + bnt_ref[...]
    t_srl = jnp.dot(fs, wnt_ref[...], preferred_element_type=_f32) + bnt_ref[...]
    a_src = jnp.dot(t_tok, w1_ref[...], preferred_element_type=_f32)
    a_dst = jnp.dot(t_srl, w2_ref[...], preferred_element_type=_f32) + batt_ref[...]
    zpad = jnp.zeros_like(a_dst[:, :H])
    for c in range(2):
        f0 = c * H
        tab1_ref[c, 0] = jnp.concatenate(
            [a_src[:, f0:f0 + H], t_tok[:, f0:f0 + H]], axis=1)
        tab1_ref[c, 1] = jnp.concatenate([a_dst[:, f0:f0 + H], zpad], axis=1)


def _tc1(ft, fs, wntT, w1T, w2T, bnt, batt):
    blk = 1000
    grid = (N // blk,)
    full = lambda shape: pl.BlockSpec(shape, lambda i: (0,) * len(shape))
    return pl.pallas_call(
        _tc1_body,
        grid=grid,
        in_specs=[
            pl.BlockSpec((blk, D), lambda i: (i, 0)),
            pl.BlockSpec((blk, D), lambda i: (i, 0)),
            full((D, D)), full((D, D)), full((D, D)),
            full((1, D)), full((1, D)),
        ],
        out_specs=pl.BlockSpec((2, 2, blk, D), lambda i: (0, 0, i, 0)),
        out_shape=jax.ShapeDtypeStruct((2, 2, N, D), _f32),
    )(ft, fs, wntT, w1T, w2T, bnt, batt)


def _sc_edge_pass(compute_edge, w_tab, acc_rows, parity):
    def body(*refs):
        if parity:
            (cidx, edges, pref, tab, out,
             bidx, didx, pbuf, grows, orows, acc, sem) = refs
        else:
            cidx, edges, tab, out, bidx, didx, grows, orows, acc, sem = refs
            pbuf = None
        c = lax.axis_index("c")
        s = lax.axis_index("s")
        _zero_vmem_rows(orows, CHUNK, D)
        _acc_zero(s, orows, acc, acc_rows)
        plsc.subcore_barrier()

        def chunk(j, carry):
            jg = s * NCHUNK + j
            pltpu.sync_copy(
                cidx.at[pl.ds(pl.multiple_of(jg * 2 * CHUNK, 8), 2 * CHUNK)],
                bidx)
            pltpu.sync_copy(
                edges.at[pl.ds(pl.multiple_of(E + jg * CHUNK, 8), CHUNK)],
                didx)
            if parity:
                pltpu.sync_copy(
                    pref.at[pl.ds(pl.multiple_of(jg * CHUNK, 8), CHUNK)],
                    pbuf)

            pltpu.async_copy(tab.at[c].at[bidx], grows, sem).wait()

            @plsc.parallel_loop(0, CHUNK, unroll=4)
            def edge(e):
                compute_edge(grows, orows, e, pbuf)
            pltpu.sync_copy(orows, acc.at[didx], add=True)
            return carry

        lax.fori_loop(0, NCHUNK, chunk, 0)
        plsc.subcore_barrier()
        _acc_copy(s, acc, out, acc_rows, dst_base=c * acc_rows)

    mesh = plsc.VectorSubcoreMesh(core_axis_name="c", subcore_axis_name="s")
    scratch = [
        pltpu.VMEM((2 * CHUNK,), jnp.int32),
        pltpu.VMEM((CHUNK,), jnp.int32),
    ]
    if parity:
        scratch.append(pltpu.VMEM((CHUNK, 16), _f32))
    scratch += [
        pltpu.VMEM((2 * CHUNK, w_tab), _f32),
        pltpu.VMEM((CHUNK, D), _f32),
        pltpu.VMEM_SHARED((acc_rows, D), _f32),
        pltpu.SemaphoreType.DMA,
    ]
    return functools.partial(
        pl.kernel,
        mesh=mesh,
        out_type=jax.ShapeDtypeStruct((2 * acc_rows, D), _f32),
        scratch_types=scratch,
    )(body)


def _sc1_edge(grows, orows, e, pbuf):
    for g in range(H // 16):
        sl = pl.ds(g * 16, 16)
        a = grows[e, sl] + grows[CHUNK + e, sl]
        a = jnp.maximum(a, a * 0.01)
        ex = jnp.exp(a)
        orows[e, sl] = ex * grows[e, pl.ds(H + g * 16, 16)]
        orows[e, pl.ds(H + g * 16, 16)] = ex


def _sc2_edge(grows, orows, e, pbuf):
    p = pbuf[e]
    q = 1.0 - p
    for g in range(H // 16):
        sl = pl.ds(g * 16, 16)
        ir = grows[e, sl]
        hr = grows[CHUNK + e, sl]
        iz = grows[e, pl.ds(H + g * 16, 16)]
        hz = grows[CHUNK + e, pl.ds(H + g * 16, 16)]
        inn = grows[e, pl.ds(2 * H + g * 16, 16)]
        hn = grows[CHUNK + e, pl.ds(2 * H + g * 16, 16)]
        hh = grows[CHUNK + e, pl.ds(3 * H + g * 16, 16)]
        r = 1.0 / (1.0 + jnp.exp(-(ir + hr)))
        z = 1.0 / (1.0 + jnp.exp(-(iz + hz)))
        x = inn + r * hn
        n = 1.0 - 2.0 / (jnp.exp(2.0 * x) + 1.0)
        m = n + z * (hh - n)
        orows[e, sl] = m * q
        orows[e, pl.ds(H + g * 16, 16)] = m * p


def _tc2_body(acc_ref, fs_ref, ft_ref, wih_ref, whh_ref, bih_ref, bhh_ref,
              hsrl_ref, tab2_ref):
    num = jnp.concatenate([acc_ref[0][:, :H], acc_ref[1][:, :H]], axis=1)
    den = jnp.concatenate([acc_ref[0][:, H:], acc_ref[1][:, H:]], axis=1)
    h = jnp.where(den > 0.0, num / den, fs_ref[...])
    hsrl_ref[...] = h
    ft = ft_ref[...]
    gi = jnp.dot(h, wih_ref[...], preferred_element_type=_f32) + bih_ref[...]
    gh = jnp.dot(ft, whh_ref[...], preferred_element_type=_f32) + bhh_ref[...]
    zpad = jnp.zeros_like(h[:, :H])
    for c in range(2):
        f0 = c * H
        tab2_ref[c, 0] = jnp.concatenate(
            [gi[:, f0:f0 + H], gi[:, D + f0:D + f0 + H],
             gi[:, 2 * D + f0:2 * D + f0 + H], zpad], axis=1)
        tab2_ref[c, 1] = jnp.concatenate(
            [gh[:, f0:f0 + H], gh[:, D + f0:D + f0 + H],
             gh[:, 2 * D + f0:2 * D + f0 + H], ft[:, f0:f0 + H]], axis=1)


def _tc2(acc1, fs, ft, wihT, whhT, bih, bhh):
    blk = 1000
    grid = (N // blk,)
    full = lambda shape: pl.BlockSpec(shape, lambda i: (0,) * len(shape))
    return pl.pallas_call(
        _tc2_body,
        grid=grid,
        in_specs=[
            pl.BlockSpec((2, blk, D), lambda i: (0, i, 0)),
            pl.BlockSpec((blk, D), lambda i: (i, 0)),
            pl.BlockSpec((blk, D), lambda i: (i, 0)),
            full((D, 3 * D)), full((D, 3 * D)),
            full((1, 3 * D)), full((1, 3 * D)),
        ],
        out_specs=[
            pl.BlockSpec((blk, D), lambda i: (i, 0)),
            pl.BlockSpec((2, 2, blk, 4 * H), lambda i: (0, 0, i, 0)),
        ],
        out_shape=[
            jax.ShapeDtypeStruct((N, D), _f32),
            jax.ShapeDtypeStruct((2, 2, N, 4 * H), _f32),
        ],
    )(acc1, fs, ft, wihT, whhT, bih, bhh)


def _tc3_body(acc_ref, wih_ref, whh_ref, bih_ref, bhh_ref, out_ref):
    h = jnp.concatenate([acc_ref[0], acc_ref[1]], axis=1)
    gi = jnp.dot(h, wih_ref[...], preferred_element_type=_f32) + bih_ref[...]
    gh = jnp.dot(h, whh_ref[...], preferred_element_type=_f32) + bhh_ref[...]
    r = jax.nn.sigmoid(gi[:, :D] + gh[:, :D])
    z = jax.nn.sigmoid(gi[:, D:2 * D] + gh[:, D:2 * D])
    n = jnp.tanh(gi[:, 2 * D:] + r * gh[:, 2 * D:])
    out_ref[...] = (1.0 - z) * n + z * h


def _tc3(acc2, wihT, whhT, bih, bhh):
    blk = 1000
    grid = (N // blk,)
    full = lambda shape: pl.BlockSpec(shape, lambda i: (0,) * len(shape))
    return pl.pallas_call(
        _tc3_body,
        grid=grid,
        in_specs=[
            pl.BlockSpec((2, blk, H), lambda i: (0, i, 0)),
            full((D, 3 * D)), full((D, 3 * D)),
            full((1, 3 * D)), full((1, 3 * D)),
        ],
        out_specs=pl.BlockSpec((blk, D), lambda i: (i, 0)),
        out_shape=jax.ShapeDtypeStruct((N, D), _f32),
    )(acc2, wihT, whhT, bih, bhh)


def _combined_idx(edges_i32):
    src = edges_i32[0].reshape(-1, CHUNK)
    dst = edges_i32[1].reshape(-1, CHUNK) + N
    return jnp.concatenate([src, dst], axis=1).reshape(-1)


def kernel(feat_tok, feat_srl, W_node_trans, b_node_trans, W_node_att,
           b_node_att, W_ih, W_hh, b_ih, b_hh, edge_tok2srl, edge_srl2tok):
    ft = feat_tok.astype(_f32)
    fs = feat_srl.astype(_f32)
    wntT = W_node_trans.astype(_f32).T
    w1T = W_node_att.astype(_f32)[:, :D].T
    w2T = W_node_att.astype(_f32)[:, D:].T
    wihT = W_ih.astype(_f32).T
    whhT = W_hh.astype(_f32).T
    bnt = b_node_trans.astype(_f32).reshape(1, D)
    batt = b_node_att.astype(_f32).reshape(1, D)
    bih = b_ih.astype(_f32).reshape(1, 3 * D)
    bhh = b_hh.astype(_f32).reshape(1, 3 * D)
    e1 = edge_tok2srl.astype(jnp.int32)
    e2 = edge_srl2tok.astype(jnp.int32)

    tab1 = _tc1(ft, fs, wntT, w1T, w2T, bnt, batt)
    sc1 = _sc_edge_pass(_sc1_edge, D, N, parity=False)
    acc1 = sc1(_combined_idx(e1), e1.reshape(-1), tab1.reshape(2, 2 * N, D))

    h_srl, tab2 = _tc2(acc1.reshape(2, N, D), fs, ft, wihT, whhT, bih, bhh)
    sc2 = _sc_edge_pass(_sc2_edge, 4 * H, N // 2, parity=True)
    e2_flat = jnp.concatenate([e2[0], e2[1] >> 1])
    par = jnp.broadcast_to((e2[1] & 1).astype(_f32)[:, None], (E, 16))
    acc2 = sc2(_combined_idx(e2), e2_flat, par, tab2.reshape(2, 2 * N, 4 * H))

    h_tok = _tc3(acc2.reshape(2, N, H), wihT, whhT, bih, bhh)
    return (h_tok, h_srl)


# speedup vs baseline: 2.0672x; 1.1943x over previous
import functools

import jax
import jax.numpy as jnp
from jax import lax
from jax.experimental import pallas as pl
from jax.experimental.pallas import tpu as pltpu
from jax.experimental.pallas import tpu_sc as plsc

N = 10000
E = 320000
D = 128
H = D // 2

NS = 16
CHUNK = 40
EPS = E // NS
NCHUNK = EPS // CHUNK
RPT = 624
RTAIL = N - NS * RPT

_f32 = jnp.float32


def _zero_vmem_rows(buf, nrows, width):
    z = jnp.zeros((16,), jnp.float32)

    def row(e, carry):
        for g in range(width // 16):
            buf[e, pl.ds(g * 16, 16)] = z
        return carry

    lax.fori_loop(0, nrows, row, 0)


def _acc_zero(s, zbuf, acc, nrows):
    rpt = (nrows // NS) // 8 * 8
    rtail = nrows - NS * rpt
    b = s * rpt
    for k in range(rpt // CHUNK):
        pltpu.sync_copy(
            zbuf.at[pl.ds(0, CHUNK)],
            acc.at[pl.ds(pl.multiple_of(b + k * CHUNK, 8), CHUNK)])
    rem = rpt - (rpt // CHUNK) * CHUNK
    if rem:
        pltpu.sync_copy(
            zbuf.at[pl.ds(0, rem)],
            acc.at[pl.ds(pl.multiple_of(b + rpt - rem, 8), rem)])

    if rtail:
        @pl.when(s == 0)
        def _tail():
            pltpu.sync_copy(zbuf.at[pl.ds(0, rtail)],
                            acc.at[pl.ds(pl.multiple_of(NS * rpt, 8), rtail)])


def _acc_copy(s, src, dst, nrows, dst_base=0):
    rpt = (nrows // NS) // 8 * 8
    rtail = nrows - NS * rpt
    b = s * rpt
    pltpu.sync_copy(src.at[pl.ds(pl.multiple_of(b, 8), rpt)],
                    dst.at[pl.ds(pl.multiple_of(dst_base + b, 8), rpt)])

    if rtail:
        @pl.when(s == 0)
        def _tail():
            t = NS * rpt
            pltpu.sync_copy(src.at[pl.ds(pl.multiple_of(t, 8), rtail)],
                            dst.at[pl.ds(pl.multiple_of(dst_base + t, 8), rtail)])


def _tc1_body(ft_ref, fs_ref, wnt_ref, w1_ref, w2_ref, bnt_ref, batt_ref,
              tab1_ref):
    ft = ft_ref[...]
    fs = fs_ref[...]
    t_tok = jnp.dot(ft, wnt_ref[...], preferred_element_type=_f32) + bnt_ref[...]
    t_srl = jnp.dot(fs, wnt_ref[...], preferred_element_type=_f32) + bnt_ref[...]
    a_src = jnp.dot(t_tok, w1_ref[...], preferred_element_type=_f32)
    a_dst = jnp.dot(t_srl, w2_ref[...], preferred_element_type=_f32) + batt_ref[...]
    zpad = jnp.zeros_like(a_dst[:, :H])
    for c in range(2):
        f0 = c * H
        tab1_ref[c, 0] = jnp.concatenate(
            [a_src[:, f0:f0 + H], t_tok[:, f0:f0 + H]], axis=1)
        tab1_ref[c, 1] = jnp.concatenate([a_dst[:, f0:f0 + H], zpad], axis=1)


def _tc1(ft, fs, wntT, w1T, w2T, bnt, batt):
    blk = 1000
    grid = (N // blk,)
    full = lambda shape: pl.BlockSpec(shape, lambda i: (0,) * len(shape))
    return pl.pallas_call(
        _tc1_body,
        grid=grid,
        in_specs=[
            pl.BlockSpec((blk, D), lambda i: (i, 0)),
            pl.BlockSpec((blk, D), lambda i: (i, 0)),
            full((D, D)), full((D, D)), full((D, D)),
            full((1, D)), full((1, D)),
        ],
        out_specs=pl.BlockSpec((2, 2, blk, D), lambda i: (0, 0, i, 0)),
        out_shape=jax.ShapeDtypeStruct((2, 2, N, D), _f32),
    )(ft, fs, wntT, w1T, w2T, bnt, batt)


def _sc_edge_pass(compute_edge, w_tab, acc_rows, parity):
    def body(*refs):
        if parity:
            (cidx, edges, pref, tab, out, bidx, didx, pbuf, grows, orows,
             acc, sem, sem_b, sem_d, sem_p) = refs
        else:
            (cidx, edges, tab, out, bidx, didx, grows, orows,
             acc, sem, sem_b, sem_d) = refs
            pbuf = None
        c = lax.axis_index("c")
        s = lax.axis_index("s")
        _zero_vmem_rows(orows, CHUNK, D)
        _acc_zero(s, orows, acc, acc_rows)
        plsc.subcore_barrier()

        def chunk(j, carry):
            jg = s * NCHUNK + j
            cp_b = pltpu.async_copy(
                cidx.at[pl.ds(pl.multiple_of(jg * 2 * CHUNK, 8), 2 * CHUNK)],
                bidx, sem_b)
            cp_d = pltpu.async_copy(
                edges.at[pl.ds(pl.multiple_of(E + jg * CHUNK, 8), CHUNK)],
                didx, sem_d)
            if parity:
                cp_p = pltpu.async_copy(
                    pref.at[pl.ds(pl.multiple_of(jg * CHUNK, 8), CHUNK)],
                    pbuf, sem_p)
                cp_p.wait()
            cp_b.wait()
            cp_d.wait()

            pltpu.async_copy(tab.at[c].at[bidx], grows, sem).wait()

            @plsc.parallel_loop(0, CHUNK, unroll=4)
            def edge(e):
                compute_edge(grows, orows, e, pbuf)
            pltpu.sync_copy(orows, acc.at[didx], add=True)
            return carry

        lax.fori_loop(0, NCHUNK, chunk, 0)
        plsc.subcore_barrier()
        _acc_copy(s, acc, out, acc_rows, dst_base=c * acc_rows)

    mesh = plsc.VectorSubcoreMesh(core_axis_name="c", subcore_axis_name="s")
    scratch = [
        pltpu.VMEM((2 * CHUNK,), jnp.int32),
        pltpu.VMEM((CHUNK,), jnp.int32),
    ]
    if parity:
        scratch.append(pltpu.VMEM((CHUNK, 16), _f32))
    scratch += [
        pltpu.VMEM((2 * CHUNK, w_tab), _f32),
        pltpu.VMEM((CHUNK, D), _f32),
        pltpu.VMEM_SHARED((acc_rows, D), _f32),
        pltpu.SemaphoreType.DMA,
        pltpu.SemaphoreType.DMA,
        pltpu.SemaphoreType.DMA,
    ]
    if parity:
        scratch.append(pltpu.SemaphoreType.DMA)
    return functools.partial(
        pl.kernel,
        mesh=mesh,
        out_type=jax.ShapeDtypeStruct((2 * acc_rows, D), _f32),
        scratch_types=scratch,
    )(body)


def _sc1_edge(grows, orows, e, pbuf):
    for g in range(H // 16):
        sl = pl.ds(g * 16, 16)
        a = grows[e, sl] + grows[CHUNK + e, sl]
        a = jnp.maximum(a, a * 0.01)
        ex = jnp.exp(a)
        orows[e, sl] = ex * grows[e, pl.ds(H + g * 16, 16)]
        orows[e, pl.ds(H + g * 16, 16)] = ex


def _sc2_edge(grows, orows, e, pbuf):
    p = pbuf[e]
    q = 1.0 - p
    for g in range(H // 16):
        sl = pl.ds(g * 16, 16)
        ir = grows[e, sl]
        hr = grows[CHUNK + e, sl]
        iz = grows[e, pl.ds(H + g * 16, 16)]
        hz = grows[CHUNK + e, pl.ds(H + g * 16, 16)]
        inn = grows[e, pl.ds(2 * H + g * 16, 16)]
        hn = grows[CHUNK + e, pl.ds(2 * H + g * 16, 16)]
        hh = grows[CHUNK + e, pl.ds(3 * H + g * 16, 16)]
        r = 1.0 / (1.0 + jnp.exp(-(ir + hr)))
        z = 1.0 / (1.0 + jnp.exp(-(iz + hz)))
        x = inn + r * hn
        n = 1.0 - 2.0 / (jnp.exp(2.0 * x) + 1.0)
        m = n + z * (hh - n)
        orows[e, sl] = m * q
        orows[e, pl.ds(H + g * 16, 16)] = m * p


def _tc2_body(acc_ref, fs_ref, ft_ref, wih_ref, whh_ref, bih_ref, bhh_ref,
              hsrl_ref, tab2_ref):
    num = jnp.concatenate([acc_ref[0][:, :H], acc_ref[1][:, :H]], axis=1)
    den = jnp.concatenate([acc_ref[0][:, H:], acc_ref[1][:, H:]], axis=1)
    h = jnp.where(den > 0.0, num / den, fs_ref[...])
    hsrl_ref[...] = h
    ft = ft_ref[...]
    gi = jnp.dot(h, wih_ref[...], preferred_element_type=_f32) + bih_ref[...]
    gh = jnp.dot(ft, whh_ref[...], preferred_element_type=_f32) + bhh_ref[...]
    zpad = jnp.zeros_like(h[:, :H])
    for c in range(2):
        f0 = c * H
        tab2_ref[c, 0] = jnp.concatenate(
            [gi[:, f0:f0 + H], gi[:, D + f0:D + f0 + H],
             gi[:, 2 * D + f0:2 * D + f0 + H], zpad], axis=1)
        tab2_ref[c, 1] = jnp.concatenate(
            [gh[:, f0:f0 + H], gh[:, D + f0:D + f0 + H],
             gh[:, 2 * D + f0:2 * D + f0 + H], ft[:, f0:f0 + H]], axis=1)


def _tc2(acc1, fs, ft, wihT, whhT, bih, bhh):
    blk = 1000
    grid = (N // blk,)
    full = lambda shape: pl.BlockSpec(shape, lambda i: (0,) * len(shape))
    return pl.pallas_call(
        _tc2_body,
        grid=grid,
        in_specs=[
            pl.BlockSpec((2, blk, D), lambda i: (0, i, 0)),
            pl.BlockSpec((blk, D), lambda i: (i, 0)),
            pl.BlockSpec((blk, D), lambda i: (i, 0)),
            full((D, 3 * D)), full((D, 3 * D)),
            full((1, 3 * D)), full((1, 3 * D)),
        ],
        out_specs=[
            pl.BlockSpec((blk, D), lambda i: (i, 0)),
            pl.BlockSpec((2, 2, blk, 4 * H), lambda i: (0, 0, i, 0)),
        ],
        out_shape=[
            jax.ShapeDtypeStruct((N, D), _f32),
            jax.ShapeDtypeStruct((2, 2, N, 4 * H), _f32),
        ],
    )(acc1, fs, ft, wihT, whhT, bih, bhh)


def _tc3_body(acc_ref, wih_ref, whh_ref, bih_ref, bhh_ref, out_ref):
    h = jnp.concatenate([acc_ref[0], acc_ref[1]], axis=1)
    gi = jnp.dot(h, wih_ref[...], preferred_element_type=_f32) + bih_ref[...]
    gh = jnp.dot(h, whh_ref[...], preferred_element_type=_f32) + bhh_ref[...]
    r = jax.nn.sigmoid(gi[:, :D] + gh[:, :D])
    z = jax.nn.sigmoid(gi[:, D:2 * D] + gh[:, D:2 * D])
    n = jnp.tanh(gi[:, 2 * D:] + r * gh[:, 2 * D:])
    out_ref[...] = (1.0 - z) * n + z * h


def _tc3(acc2, wihT, whhT, bih, bhh):
    blk = 1000
    grid = (N // blk,)
    full = lambda shape: pl.BlockSpec(shape, lambda i: (0,) * len(shape))
    return pl.pallas_call(
        _tc3_body,
        grid=grid,
        in_specs=[
            pl.BlockSpec((2, blk, H), lambda i: (0, i, 0)),
            full((D, 3 * D)), full((D, 3 * D)),
            full((1, 3 * D)), full((1, 3 * D)),
        ],
        out_specs=pl.BlockSpec((blk, D), lambda i: (i, 0)),
        out_shape=jax.ShapeDtypeStruct((N, D), _f32),
    )(acc2, wihT, whhT, bih, bhh)


def _combined_idx(edges_i32):
    src = edges_i32[0].reshape(-1, CHUNK)
    dst = edges_i32[1].reshape(-1, CHUNK) + N
    return jnp.concatenate([src, dst], axis=1).reshape(-1)


def kernel(feat_tok, feat_srl, W_node_trans, b_node_trans, W_node_att,
           b_node_att, W_ih, W_hh, b_ih, b_hh, edge_tok2srl, edge_srl2tok):
    ft = feat_tok.astype(_f32)
    fs = feat_srl.astype(_f32)
    wntT = W_node_trans.astype(_f32).T
    w1T = W_node_att.astype(_f32)[:, :D].T
    w2T = W_node_att.astype(_f32)[:, D:].T
    wihT = W_ih.astype(_f32).T
    whhT = W_hh.astype(_f32).T
    bnt = b_node_trans.astype(_f32).reshape(1, D)
    batt = b_node_att.astype(_f32).reshape(1, D)
    bih = b_ih.astype(_f32).reshape(1, 3 * D)
    bhh = b_hh.astype(_f32).reshape(1, 3 * D)
    e1 = edge_tok2srl.astype(jnp.int32)
    e2 = edge_srl2tok.astype(jnp.int32)

    tab1 = _tc1(ft, fs, wntT, w1T, w2T, bnt, batt)
    sc1 = _sc_edge_pass(_sc1_edge, D, N, parity=False)
    acc1 = sc1(_combined_idx(e1), e1.reshape(-1), tab1.reshape(2, 2 * N, D))

    h_srl, tab2 = _tc2(acc1.reshape(2, N, D), fs, ft, wihT, whhT, bih, bhh)
    sc2 = _sc_edge_pass(_sc2_edge, 4 * H, N // 2, parity=True)
    e2_flat = jnp.concatenate([e2[0], e2[1] >> 1])
    par = jnp.broadcast_to((e2[1] & 1).astype(_f32)[:, None], (E, 16))
    acc2 = sc2(_combined_idx(e2), e2_flat, par, tab2.reshape(2, 2 * N, 4 * H))

    h_tok = _tc3(acc2.reshape(2, N, H), wihT, whhT, bih, bhh)
    return (h_tok, h_srl)


# speedup vs baseline: 2.2312x; 1.0793x over previous
import functools

import jax
import jax.numpy as jnp
from jax import lax
from jax.experimental import pallas as pl
from jax.experimental.pallas import tpu as pltpu
from jax.experimental.pallas import tpu_sc as plsc

N = 10000
E = 320000
D = 128
H = D // 2

NS = 16
CHUNK = 40
EPS = E // NS
NCHUNK = EPS // CHUNK
RPT = 624
RTAIL = N - NS * RPT

_f32 = jnp.float32


def _zero_vmem_rows(buf, nrows, width):
    z = jnp.zeros((16,), jnp.float32)

    def row(e, carry):
        for g in range(width // 16):
            buf[e, pl.ds(g * 16, 16)] = z
        return carry

    lax.fori_loop(0, nrows, row, 0)


def _acc_zero(s, zbuf, acc, nrows):
    rpt = (nrows // NS) // 8 * 8
    rtail = nrows - NS * rpt
    b = s * rpt
    for k in range(rpt // CHUNK):
        pltpu.sync_copy(
            zbuf.at[pl.ds(0, CHUNK)],
            acc.at[pl.ds(pl.multiple_of(b + k * CHUNK, 8), CHUNK)])
    rem = rpt - (rpt // CHUNK) * CHUNK
    if rem:
        pltpu.sync_copy(
            zbuf.at[pl.ds(0, rem)],
            acc.at[pl.ds(pl.multiple_of(b + rpt - rem, 8), rem)])

    if rtail:
        @pl.when(s == 0)
        def _tail():
            pltpu.sync_copy(zbuf.at[pl.ds(0, rtail)],
                            acc.at[pl.ds(pl.multiple_of(NS * rpt, 8), rtail)])


def _acc_copy(s, src, dst, nrows, dst_base=0):
    rpt = (nrows // NS) // 8 * 8
    rtail = nrows - NS * rpt
    b = s * rpt
    pltpu.sync_copy(src.at[pl.ds(pl.multiple_of(b, 8), rpt)],
                    dst.at[pl.ds(pl.multiple_of(dst_base + b, 8), rpt)])

    if rtail:
        @pl.when(s == 0)
        def _tail():
            t = NS * rpt
            pltpu.sync_copy(src.at[pl.ds(pl.multiple_of(t, 8), rtail)],
                            dst.at[pl.ds(pl.multiple_of(dst_base + t, 8), rtail)])


def _tc1_body(ft_ref, fs_ref, wnt_ref, w1_ref, w2_ref, bnt_ref, batt_ref,
              tab1_ref):
    ft = ft_ref[...]
    fs = fs_ref[...]
    t_tok = jnp.dot(ft, wnt_ref[...], preferred_element_type=_f32) + bnt_ref[...]
    t_srl = jnp.dot(fs, wnt_ref[...], preferred_element_type=_f32) + bnt_ref[...]
    a_src = jnp.dot(t_tok, w1_ref[...], preferred_element_type=_f32)
    a_dst = jnp.dot(t_srl, w2_ref[...], preferred_element_type=_f32) + batt_ref[...]
    zpad = jnp.zeros_like(a_dst[:, :H])
    for c in range(2):
        f0 = c * H
        tab1_ref[c, 0] = jnp.concatenate(
            [a_src[:, f0:f0 + H], t_tok[:, f0:f0 + H]], axis=1)
        tab1_ref[c, 1] = jnp.concatenate([a_dst[:, f0:f0 + H], zpad], axis=1)


def _tc1(ft, fs, wntT, w1T, w2T, bnt, batt):
    blk = 1000
    grid = (N // blk,)
    full = lambda shape: pl.BlockSpec(shape, lambda i: (0,) * len(shape))
    return pl.pallas_call(
        _tc1_body,
        grid=grid,
        in_specs=[
            pl.BlockSpec((blk, D), lambda i: (i, 0)),
            pl.BlockSpec((blk, D), lambda i: (i, 0)),
            full((D, D)), full((D, D)), full((D, D)),
            full((1, D)), full((1, D)),
        ],
        out_specs=pl.BlockSpec((2, 2, blk, D), lambda i: (0, 0, i, 0)),
        out_shape=jax.ShapeDtypeStruct((2, 2, N, D), _f32),
    )(ft, fs, wntT, w1T, w2T, bnt, batt)


def _sc_edge_pass(compute_edge, w_tab, acc_rows, parity):
    def body(*refs):
        if parity:
            (cidx, edges, pref, tab, out, bidx, didx, pbuf, grows, orows,
             acc, sem, sem_b, sem_d, sem_s, sem_p) = refs
        else:
            (cidx, edges, tab, out, bidx, didx, grows, orows,
             acc, sem, sem_b, sem_d, sem_s) = refs
            pbuf = None
        c = lax.axis_index("c")
        s = lax.axis_index("s")
        _zero_vmem_rows(orows, 2 * CHUNK, D)
        _acc_zero(s, orows, acc, acc_rows)
        plsc.subcore_barrier()

        def chunk(j, carry):
            jg = s * NCHUNK + j
            phase = jnp.bitwise_and(j, 1)
            nxt = 1 - phase
            obase = pl.multiple_of(phase * CHUNK, 8)
            cp_b = pltpu.async_copy(
                cidx.at[pl.ds(pl.multiple_of(jg * 2 * CHUNK, 8), 2 * CHUNK)],
                bidx, sem_b)
            cp_d = pltpu.async_copy(
                edges.at[pl.ds(pl.multiple_of(E + jg * CHUNK, 8), CHUNK)],
                didx.at[phase], sem_d)
            if parity:
                cp_p = pltpu.async_copy(
                    pref.at[pl.ds(pl.multiple_of(jg * CHUNK, 8), CHUNK)],
                    pbuf.at[phase], sem_p)
                cp_p.wait()
            cp_b.wait()
            cp_d.wait()

            pltpu.async_copy(tab.at[c].at[bidx], grows, sem).wait()

            @plsc.parallel_loop(0, CHUNK, unroll=4)
            def edge(e):
                compute_edge(grows, orows, e, pbuf, obase, phase)

            @pl.when(j > 0)
            def _drain():
                pltpu.make_async_copy(
                    orows.at[pl.ds(pl.multiple_of(nxt * CHUNK, 8), CHUNK)],
                    acc.at[didx.at[nxt]], sem_s).wait()

            pltpu.async_copy(orows.at[pl.ds(obase, CHUNK)],
                             acc.at[didx.at[phase]], sem_s, add=True)
            return carry

        lax.fori_loop(0, NCHUNK, chunk, 0)
        pltpu.make_async_copy(orows.at[pl.ds(CHUNK, CHUNK)],
                              acc.at[didx.at[1]], sem_s).wait()
        plsc.subcore_barrier()
        _acc_copy(s, acc, out, acc_rows, dst_base=c * acc_rows)

    mesh = plsc.VectorSubcoreMesh(core_axis_name="c", subcore_axis_name="s")
    scratch = [
        pltpu.VMEM((2 * CHUNK,), jnp.int32),
        pltpu.VMEM((2, CHUNK), jnp.int32),
    ]
    if parity:
        scratch.append(pltpu.VMEM((2, CHUNK, 16), _f32))
    scratch += [
        pltpu.VMEM((2 * CHUNK, w_tab), _f32),
        pltpu.VMEM((2 * CHUNK, D), _f32),
        pltpu.VMEM_SHARED((acc_rows, D), _f32),
        pltpu.SemaphoreType.DMA,
        pltpu.SemaphoreType.DMA,
        pltpu.SemaphoreType.DMA,
        pltpu.SemaphoreType.DMA,
    ]
    if parity:
        scratch.append(pltpu.SemaphoreType.DMA)
    return functools.partial(
        pl.kernel,
        mesh=mesh,
        out_type=jax.ShapeDtypeStruct((2 * acc_rows, D), _f32),
        scratch_types=scratch,
    )(body)


def _sc1_edge(grows, orows, e, pbuf, obase, phase):
    for g in range(H // 16):
        sl = pl.ds(g * 16, 16)
        a = grows[e, sl] + grows[CHUNK + e, sl]
        a = jnp.maximum(a, a * 0.01)
        ex = jnp.exp(a)
        orows[obase + e, sl] = ex * grows[e, pl.ds(H + g * 16, 16)]
        orows[obase + e, pl.ds(H + g * 16, 16)] = ex


def _sc2_edge(grows, orows, e, pbuf, obase, phase):
    p = pbuf[phase, e]
    q = 1.0 - p
    for g in range(H // 16):
        sl = pl.ds(g * 16, 16)
        ir = grows[e, sl]
        hr = grows[CHUNK + e, sl]
        iz = grows[e, pl.ds(H + g * 16, 16)]
        hz = grows[CHUNK + e, pl.ds(H + g * 16, 16)]
        inn = grows[e, pl.ds(2 * H + g * 16, 16)]
        hn = grows[CHUNK + e, pl.ds(2 * H + g * 16, 16)]
        hh = grows[CHUNK + e, pl.ds(3 * H + g * 16, 16)]
        r = 1.0 / (1.0 + jnp.exp(-(ir + hr)))
        z = 1.0 / (1.0 + jnp.exp(-(iz + hz)))
        x = inn + r * hn
        n = 1.0 - 2.0 / (jnp.exp(2.0 * x) + 1.0)
        m = n + z * (hh - n)
        orows[obase + e, sl] = m * q
        orows[obase + e, pl.ds(H + g * 16, 16)] = m * p


def _tc2_body(acc_ref, fs_ref, ft_ref, wih_ref, whh_ref, bih_ref, bhh_ref,
              hsrl_ref, tab2_ref):
    num = jnp.concatenate([acc_ref[0][:, :H], acc_ref[1][:, :H]], axis=1)
    den = jnp.concatenate([acc_ref[0][:, H:], acc_ref[1][:, H:]], axis=1)
    h = jnp.where(den > 0.0, num / den, fs_ref[...])
    hsrl_ref[...] = h
    ft = ft_ref[...]
    gi = jnp.dot(h, wih_ref[...], preferred_element_type=_f32) + bih_ref[...]
    gh = jnp.dot(ft, whh_ref[...], preferred_element_type=_f32) + bhh_ref[...]
    zpad = jnp.zeros_like(h[:, :H])
    for c in range(2):
        f0 = c * H
        tab2_ref[c, 0] = jnp.concatenate(
            [gi[:, f0:f0 + H], gi[:, D + f0:D + f0 + H],
             gi[:, 2 * D + f0:2 * D + f0 + H], zpad], axis=1)
        tab2_ref[c, 1] = jnp.concatenate(
            [gh[:, f0:f0 + H], gh[:, D + f0:D + f0 + H],
             gh[:, 2 * D + f0:2 * D + f0 + H], ft[:, f0:f0 + H]], axis=1)


def _tc2(acc1, fs, ft, wihT, whhT, bih, bhh):
    blk = 1000
    grid = (N // blk,)
    full = lambda shape: pl.BlockSpec(shape, lambda i: (0,) * len(shape))
    return pl.pallas_call(
        _tc2_body,
        grid=grid,
        in_specs=[
            pl.BlockSpec((2, blk, D), lambda i: (0, i, 0)),
            pl.BlockSpec((blk, D), lambda i: (i, 0)),
            pl.BlockSpec((blk, D), lambda i: (i, 0)),
            full((D, 3 * D)), full((D, 3 * D)),
            full((1, 3 * D)), full((1, 3 * D)),
        ],
        out_specs=[
            pl.BlockSpec((blk, D), lambda i: (i, 0)),
            pl.BlockSpec((2, 2, blk, 4 * H), lambda i: (0, 0, i, 0)),
        ],
        out_shape=[
            jax.ShapeDtypeStruct((N, D), _f32),
            jax.ShapeDtypeStruct((2, 2, N, 4 * H), _f32),
        ],
    )(acc1, fs, ft, wihT, whhT, bih, bhh)


def _tc3_body(acc_ref, wih_ref, whh_ref, bih_ref, bhh_ref, out_ref):
    h = jnp.concatenate([acc_ref[0], acc_ref[1]], axis=1)
    gi = jnp.dot(h, wih_ref[...], preferred_element_type=_f32) + bih_ref[...]
    gh = jnp.dot(h, whh_ref[...], preferred_element_type=_f32) + bhh_ref[...]
    r = jax.nn.sigmoid(gi[:, :D] + gh[:, :D])
    z = jax.nn.sigmoid(gi[:, D:2 * D] + gh[:, D:2 * D])
    n = jnp.tanh(gi[:, 2 * D:] + r * gh[:, 2 * D:])
    out_ref[...] = (1.0 - z) * n + z * h


def _tc3(acc2, wihT, whhT, bih, bhh):
    blk = 1000
    grid = (N // blk,)
    full = lambda shape: pl.BlockSpec(shape, lambda i: (0,) * len(shape))
    return pl.pallas_call(
        _tc3_body,
        grid=grid,
        in_specs=[
            pl.BlockSpec((2, blk, H), lambda i: (0, i, 0)),
            full((D, 3 * D)), full((D, 3 * D)),
            full((1, 3 * D)), full((1, 3 * D)),
        ],
        out_specs=pl.BlockSpec((blk, D), lambda i: (i, 0)),
        out_shape=jax.ShapeDtypeStruct((N, D), _f32),
    )(acc2, wihT, whhT, bih, bhh)


def _combined_idx(edges_i32):
    src = edges_i32[0].reshape(-1, CHUNK)
    dst = edges_i32[1].reshape(-1, CHUNK) + N
    return jnp.concatenate([src, dst], axis=1).reshape(-1)


def kernel(feat_tok, feat_srl, W_node_trans, b_node_trans, W_node_att,
           b_node_att, W_ih, W_hh, b_ih, b_hh, edge_tok2srl, edge_srl2tok):
    ft = feat_tok.astype(_f32)
    fs = feat_srl.astype(_f32)
    wntT = W_node_trans.astype(_f32).T
    w1T = W_node_att.astype(_f32)[:, :D].T
    w2T = W_node_att.astype(_f32)[:, D:].T
    wihT = W_ih.astype(_f32).T
    whhT = W_hh.astype(_f32).T
    bnt = b_node_trans.astype(_f32).reshape(1, D)
    batt = b_node_att.astype(_f32).reshape(1, D)
    bih = b_ih.astype(_f32).reshape(1, 3 * D)
    bhh = b_hh.astype(_f32).reshape(1, 3 * D)
    e1 = edge_tok2srl.astype(jnp.int32)
    e2 = edge_srl2tok.astype(jnp.int32)

    tab1 = _tc1(ft, fs, wntT, w1T, w2T, bnt, batt)
    sc1 = _sc_edge_pass(_sc1_edge, D, N, parity=False)
    acc1 = sc1(_combined_idx(e1), e1.reshape(-1), tab1.reshape(2, 2 * N, D))

    h_srl, tab2 = _tc2(acc1.reshape(2, N, D), fs, ft, wihT, whhT, bih, bhh)
    sc2 = _sc_edge_pass(_sc2_edge, 4 * H, N // 2, parity=True)
    e2_flat = jnp.concatenate([e2[0], e2[1] >> 1])
    par = jnp.broadcast_to((e2[1] & 1).astype(_f32)[:, None], (E, 16))
    acc2 = sc2(_combined_idx(e2), e2_flat, par, tab2.reshape(2, 2 * N, 4 * H))

    h_tok = _tc3(acc2.reshape(2, N, H), wihT, whhT, bih, bhh)
    return (h_tok, h_srl)


# speedup vs baseline: 3.5385x; 1.5859x over previous
import functools

import jax
import jax.numpy as jnp
from jax import lax
from jax.experimental import pallas as pl
from jax.experimental.pallas import tpu as pltpu
from jax.experimental.pallas import tpu_sc as plsc

N = 10000
E = 320000
D = 128
H = D // 2

NS = 16
CHUNK = 40
EPS = E // NS
NCHUNK = EPS // CHUNK
RPT = 624
RTAIL = N - NS * RPT

_f32 = jnp.float32


def _zero_vmem_rows(buf, nrows, width):
    z = jnp.zeros((16,), jnp.float32)

    def row(e, carry):
        for g in range(width // 16):
            buf[e, pl.ds(g * 16, 16)] = z
        return carry

    lax.fori_loop(0, nrows, row, 0)


def _acc_zero(s, zbuf, acc, nrows):
    rpt = (nrows // NS) // 8 * 8
    rtail = nrows - NS * rpt
    b = s * rpt
    for k in range(rpt // CHUNK):
        pltpu.sync_copy(
            zbuf.at[pl.ds(0, CHUNK)],
            acc.at[pl.ds(pl.multiple_of(b + k * CHUNK, 8), CHUNK)])
    rem = rpt - (rpt // CHUNK) * CHUNK
    if rem:
        pltpu.sync_copy(
            zbuf.at[pl.ds(0, rem)],
            acc.at[pl.ds(pl.multiple_of(b + rpt - rem, 8), rem)])

    if rtail:
        @pl.when(s == 0)
        def _tail():
            pltpu.sync_copy(zbuf.at[pl.ds(0, rtail)],
                            acc.at[pl.ds(pl.multiple_of(NS * rpt, 8), rtail)])


def _acc_copy(s, src, dst, nrows, dst_base=0):
    rpt = (nrows // NS) // 8 * 8
    rtail = nrows - NS * rpt
    b = s * rpt
    pltpu.sync_copy(src.at[pl.ds(pl.multiple_of(b, 8), rpt)],
                    dst.at[pl.ds(pl.multiple_of(dst_base + b, 8), rpt)])

    if rtail:
        @pl.when(s == 0)
        def _tail():
            t = NS * rpt
            pltpu.sync_copy(src.at[pl.ds(pl.multiple_of(t, 8), rtail)],
                            dst.at[pl.ds(pl.multiple_of(dst_base + t, 8), rtail)])


def _tc1_body(ft_ref, fs_ref, wnt_ref, w1_ref, w2_ref, bnt_ref, batt_ref,
              tab1_ref):
    ft = ft_ref[...]
    fs = fs_ref[...]
    t_tok = jnp.dot(ft, wnt_ref[...], preferred_element_type=_f32) + bnt_ref[...]
    t_srl = jnp.dot(fs, wnt_ref[...], preferred_element_type=_f32) + bnt_ref[...]
    a_src = jnp.dot(t_tok, w1_ref[...], preferred_element_type=_f32)
    a_dst = jnp.dot(t_srl, w2_ref[...], preferred_element_type=_f32) + batt_ref[...]
    zpad = jnp.zeros_like(a_dst[:, :H])
    for c in range(2):
        f0 = c * H
        tab1_ref[c, 0] = jnp.concatenate(
            [a_src[:, f0:f0 + H], t_tok[:, f0:f0 + H]], axis=1)
        tab1_ref[c, 1] = jnp.concatenate([a_dst[:, f0:f0 + H], zpad], axis=1)


def _tc1(ft, fs, wntT, w1T, w2T, bnt, batt):
    blk = 1000
    grid = (N // blk,)
    full = lambda shape: pl.BlockSpec(shape, lambda i: (0,) * len(shape))
    return pl.pallas_call(
        _tc1_body,
        grid=grid,
        in_specs=[
            pl.BlockSpec((blk, D), lambda i: (i, 0)),
            pl.BlockSpec((blk, D), lambda i: (i, 0)),
            full((D, D)), full((D, D)), full((D, D)),
            full((1, D)), full((1, D)),
        ],
        out_specs=pl.BlockSpec((2, 2, blk, D), lambda i: (0, 0, i, 0)),
        out_shape=jax.ShapeDtypeStruct((2, 2, N, D), _f32),
    )(ft, fs, wntT, w1T, w2T, bnt, batt)


def _sc_edge_pass(compute_edge, w_tab, acc_rows, parity):
    def body(*refs):
        if parity:
            (cidx, edges, pref, tab, out, bidx, didx, pbuf, grows, orows,
             acc, sem, sem_b, sem_d, sem_s, sem_p) = refs
        else:
            (cidx, edges, tab, out, bidx, didx, grows, orows,
             acc, sem, sem_b, sem_d, sem_s) = refs
            pbuf = None
        c = lax.axis_index("c")
        s = lax.axis_index("s")
        _zero_vmem_rows(orows, 2 * CHUNK, D)
        _acc_zero(s, orows, acc, acc_rows)
        plsc.subcore_barrier()

        def chunk(j, carry):
            jg = s * NCHUNK + j
            jn = jg + 1
            phase = jnp.bitwise_and(j, 1)
            nphase = jnp.bitwise_and(j + 1, 1)
            obase = pl.multiple_of(phase * CHUNK, 8)
            gbase = pl.multiple_of(phase * 2 * CHUNK, 8)
            ngbase = pl.multiple_of(nphase * 2 * CHUNK, 8)
            has_next = j < NCHUNK - 1

            @pl.when(has_next)
            def _prefetch_loads():
                pltpu.async_copy(
                    cidx.at[pl.ds(pl.multiple_of(jn * 2 * CHUNK, 8),
                                  2 * CHUNK)],
                    bidx.at[nphase], sem_b)
                pltpu.async_copy(
                    edges.at[pl.ds(pl.multiple_of(E + jn * CHUNK, 8), CHUNK)],
                    didx.at[jnp.bitwise_and(j + 1, 3)], sem_d)
                if parity:
                    pltpu.async_copy(
                        pref.at[pl.ds(pl.multiple_of(jn * CHUNK, 8), CHUNK)],
                        pbuf.at[nphase], sem_p)

            @pl.when(j >= 0)
            def _wait_gather():
                pltpu.make_async_copy(
                    tab.at[c].at[bidx.at[phase]],
                    grows.at[pl.ds(gbase, 2 * CHUNK)], sem).wait()

            @pl.when(has_next)
            def _issue_gather():
                pltpu.make_async_copy(
                    cidx.at[pl.ds(pl.multiple_of(jn * 2 * CHUNK, 8),
                                  2 * CHUNK)],
                    bidx.at[nphase], sem_b).wait()
                pltpu.make_async_copy(
                    edges.at[pl.ds(pl.multiple_of(E + jn * CHUNK, 8), CHUNK)],
                    didx.at[jnp.bitwise_and(j + 1, 3)], sem_d).wait()
                if parity:
                    pltpu.make_async_copy(
                        pref.at[pl.ds(pl.multiple_of(jn * CHUNK, 8), CHUNK)],
                        pbuf.at[nphase], sem_p).wait()
                pltpu.async_copy(tab.at[c].at[bidx.at[nphase]],
                                 grows.at[pl.ds(ngbase, 2 * CHUNK)], sem)

            @pl.when(j >= 0)
            def _compute_scatter():
                @plsc.parallel_loop(0, CHUNK, unroll=4)
                def edge(e):
                    compute_edge(grows, orows, e, pbuf, obase, gbase, phase)

                @pl.when(j > 0)
                def _drain():
                    pltpu.make_async_copy(
                        orows.at[pl.ds(pl.multiple_of((1 - phase) * CHUNK, 8),
                                       CHUNK)],
                        acc.at[didx.at[jnp.bitwise_and(j - 1, 3)]],
                        sem_s).wait()

                pltpu.async_copy(orows.at[pl.ds(obase, CHUNK)],
                                 acc.at[didx.at[jnp.bitwise_and(j, 3)]],
                                 sem_s, add=True)
            return carry

        lax.fori_loop(-1, NCHUNK, chunk, 0)
        pltpu.make_async_copy(
            orows.at[pl.ds(CHUNK, CHUNK)],
            acc.at[didx.at[(NCHUNK - 1) & 3]], sem_s).wait()
        plsc.subcore_barrier()
        _acc_copy(s, acc, out, acc_rows, dst_base=c * acc_rows)

    mesh = plsc.VectorSubcoreMesh(core_axis_name="c", subcore_axis_name="s")
    scratch = [
        pltpu.VMEM((2, 2 * CHUNK), jnp.int32),
        pltpu.VMEM((4, CHUNK), jnp.int32),
    ]
    if parity:
        scratch.append(pltpu.VMEM((2, CHUNK, 16), _f32))
    scratch += [
        pltpu.VMEM((4 * CHUNK, w_tab), _f32),
        pltpu.VMEM((2 * CHUNK, D), _f32),
        pltpu.VMEM_SHARED((acc_rows, D), _f32),
        pltpu.SemaphoreType.DMA,
        pltpu.SemaphoreType.DMA,
        pltpu.SemaphoreType.DMA,
        pltpu.SemaphoreType.DMA,
    ]
    if parity:
        scratch.append(pltpu.SemaphoreType.DMA)
    return functools.partial(
        pl.kernel,
        mesh=mesh,
        out_type=jax.ShapeDtypeStruct((2 * acc_rows, D), _f32),
        scratch_types=scratch,
    )(body)


def _sc1_edge(grows, orows, e, pbuf, obase, gbase, phase):
    for g in range(H // 16):
        sl = pl.ds(g * 16, 16)
        a = grows[gbase + e, sl] + grows[gbase + CHUNK + e, sl]
        a = jnp.maximum(a, a * 0.01)
        ex = jnp.exp(a)
        orows[obase + e, sl] = ex * grows[gbase + e, pl.ds(H + g * 16, 16)]
        orows[obase + e, pl.ds(H + g * 16, 16)] = ex


def _sc2_edge(grows, orows, e, pbuf, obase, gbase, phase):
    p = pbuf[phase, e]
    q = 1.0 - p
    for g in range(H // 16):
        sl = pl.ds(g * 16, 16)
        ir = grows[gbase + e, sl]
        hr = grows[gbase + CHUNK + e, sl]
        iz = grows[gbase + e, pl.ds(H + g * 16, 16)]
        hz = grows[gbase + CHUNK + e, pl.ds(H + g * 16, 16)]
        inn = grows[gbase + e, pl.ds(2 * H + g * 16, 16)]
        hn = grows[gbase + CHUNK + e, pl.ds(2 * H + g * 16, 16)]
        hh = grows[gbase + CHUNK + e, pl.ds(3 * H + g * 16, 16)]
        r = 1.0 / (1.0 + jnp.exp(-(ir + hr)))
        z = 1.0 / (1.0 + jnp.exp(-(iz + hz)))
        x = inn + r * hn
        n = 1.0 - 2.0 / (jnp.exp(2.0 * x) + 1.0)
        m = n + z * (hh - n)
        orows[obase + e, sl] = m * q
        orows[obase + e, pl.ds(H + g * 16, 16)] = m * p


def _tc2_body(acc_ref, fs_ref, ft_ref, wih_ref, whh_ref, bih_ref, bhh_ref,
              hsrl_ref, tab2_ref):
    num = jnp.concatenate([acc_ref[0][:, :H], acc_ref[1][:, :H]], axis=1)
    den = jnp.concatenate([acc_ref[0][:, H:], acc_ref[1][:, H:]], axis=1)
    h = jnp.where(den > 0.0, num / den, fs_ref[...])
    hsrl_ref[...] = h
    ft = ft_ref[...]
    gi = jnp.dot(h, wih_ref[...], preferred_element_type=_f32) + bih_ref[...]
    gh = jnp.dot(ft, whh_ref[...], preferred_element_type=_f32) + bhh_ref[...]
    zpad = jnp.zeros_like(h[:, :H])
    for c in range(2):
        f0 = c * H
        tab2_ref[c, 0] = jnp.concatenate(
            [gi[:, f0:f0 + H], gi[:, D + f0:D + f0 + H],
             gi[:, 2 * D + f0:2 * D + f0 + H], zpad], axis=1)
        tab2_ref[c, 1] = jnp.concatenate(
            [gh[:, f0:f0 + H], gh[:, D + f0:D + f0 + H],
             gh[:, 2 * D + f0:2 * D + f0 + H], ft[:, f0:f0 + H]], axis=1)


def _tc2(acc1, fs, ft, wihT, whhT, bih, bhh):
    blk = 1000
    grid = (N // blk,)
    full = lambda shape: pl.BlockSpec(shape, lambda i: (0,) * len(shape))
    return pl.pallas_call(
        _tc2_body,
        grid=grid,
        in_specs=[
            pl.BlockSpec((2, blk, D), lambda i: (0, i, 0)),
            pl.BlockSpec((blk, D), lambda i: (i, 0)),
            pl.BlockSpec((blk, D), lambda i: (i, 0)),
            full((D, 3 * D)), full((D, 3 * D)),
            full((1, 3 * D)), full((1, 3 * D)),
        ],
        out_specs=[
            pl.BlockSpec((blk, D), lambda i: (i, 0)),
            pl.BlockSpec((2, 2, blk, 4 * H), lambda i: (0, 0, i, 0)),
        ],
        out_shape=[
            jax.ShapeDtypeStruct((N, D), _f32),
            jax.ShapeDtypeStruct((2, 2, N, 4 * H), _f32),
        ],
    )(acc1, fs, ft, wihT, whhT, bih, bhh)


def _tc3_body(acc_ref, wih_ref, whh_ref, bih_ref, bhh_ref, out_ref):
    h = jnp.concatenate([acc_ref[0], acc_ref[1]], axis=1)
    gi = jnp.dot(h, wih_ref[...], preferred_element_type=_f32) + bih_ref[...]
    gh = jnp.dot(h, whh_ref[...], preferred_element_type=_f32) + bhh_ref[...]
    r = jax.nn.sigmoid(gi[:, :D] + gh[:, :D])
    z = jax.nn.sigmoid(gi[:, D:2 * D] + gh[:, D:2 * D])
    n = jnp.tanh(gi[:, 2 * D:] + r * gh[:, 2 * D:])
    out_ref[...] = (1.0 - z) * n + z * h


def _tc3(acc2, wihT, whhT, bih, bhh):
    blk = 1000
    grid = (N // blk,)
    full = lambda shape: pl.BlockSpec(shape, lambda i: (0,) * len(shape))
    return pl.pallas_call(
        _tc3_body,
        grid=grid,
        in_specs=[
            pl.BlockSpec((2, blk, H), lambda i: (0, i, 0)),
            full((D, 3 * D)), full((D, 3 * D)),
            full((1, 3 * D)), full((1, 3 * D)),
        ],
        out_specs=pl.BlockSpec((blk, D), lambda i: (i, 0)),
        out_shape=jax.ShapeDtypeStruct((N, D), _f32),
    )(acc2, wihT, whhT, bih, bhh)


def _combined_idx(edges_i32):
    src = edges_i32[0].reshape(-1, CHUNK)
    dst = edges_i32[1].reshape(-1, CHUNK) + N
    return jnp.concatenate([src, dst], axis=1).reshape(-1)


def kernel(feat_tok, feat_srl, W_node_trans, b_node_trans, W_node_att,
           b_node_att, W_ih, W_hh, b_ih, b_hh, edge_tok2srl, edge_srl2tok):
    ft = feat_tok.astype(_f32)
    fs = feat_srl.astype(_f32)
    wntT = W_node_trans.astype(_f32).T
    w1T = W_node_att.astype(_f32)[:, :D].T
    w2T = W_node_att.astype(_f32)[:, D:].T
    wihT = W_ih.astype(_f32).T
    whhT = W_hh.astype(_f32).T
    bnt = b_node_trans.astype(_f32).reshape(1, D)
    batt = b_node_att.astype(_f32).reshape(1, D)
    bih = b_ih.astype(_f32).reshape(1, 3 * D)
    bhh = b_hh.astype(_f32).reshape(1, 3 * D)
    e1 = edge_tok2srl.astype(jnp.int32)
    e2 = edge_srl2tok.astype(jnp.int32)

    tab1 = _tc1(ft, fs, wntT, w1T, w2T, bnt, batt)
    sc1 = _sc_edge_pass(_sc1_edge, D, N, parity=False)
    acc1 = sc1(_combined_idx(e1), e1.reshape(-1), tab1.reshape(2, 2 * N, D))

    h_srl, tab2 = _tc2(acc1.reshape(2, N, D), fs, ft, wihT, whhT, bih, bhh)
    sc2 = _sc_edge_pass(_sc2_edge, 4 * H, N // 2, parity=True)
    e2_flat = jnp.concatenate([e2[0], e2[1] >> 1])
    par = jnp.broadcast_to((e2[1] & 1).astype(_f32)[:, None], (E, 16))
    acc2 = sc2(_combined_idx(e2), e2_flat, par, tab2.reshape(2, 2 * N, 4 * H))

    h_tok = _tc3(acc2.reshape(2, N, H), wihT, whhT, bih, bhh)
    return (h_tok, h_srl)
